# baseline (speedup 1.0000x reference)
import functools

import jax
import jax.numpy as jnp
from jax import lax
from jax.experimental import pallas as pl
from jax.experimental.pallas import tpu as pltpu
from jax.experimental.pallas import tpu_sc as plsc

EPS = 1e-8
_NW = 32
_CHUNK = 512
_GROUPS = _CHUNK // 16
_NS = _CHUNK // 128


def _rsqrt(x):
    y = plsc.bitcast(jnp.int32(0x5F3759DF) - (plsc.bitcast(x, jnp.int32) >> 1),
                     jnp.float32)
    y = y * (1.5 - 0.5 * x * y * y)
    y = y * (1.5 - 0.5 * x * y * y)
    y = y * (1.5 - 0.5 * x * y * y)
    return y


def _sin01(x, t):
    s = 2.7557319e-6 * t - 1.9841270e-4
    s = s * t + 8.3333333e-3
    s = s * t - 1.6666667e-1
    s = s * t + 1.0
    return x * s


def _cos01(t):
    c = -2.7557319e-7 * t + 2.4801587e-5
    c = c * t - 1.3888889e-3
    c = c * t + 4.1666667e-2
    c = c * t - 5.0e-1
    return c * t + 1.0


def _make_fused(N):
    per_w = N // _NW
    chunks = per_w // _CHUNK
    mesh = plsc.VectorSubcoreMesh(core_axis_name="c", subcore_axis_name="s")
    two = lambda shape, dt: [pltpu.VMEM(shape, dt), pltpu.VMEM(shape, dt)]

    @functools.partial(
        pl.kernel,
        mesh=mesh,
        out_type=(
            jax.ShapeDtypeStruct((N, 4), jnp.float32),
            jax.ShapeDtypeStruct((N, 32), jnp.float32),
            jax.ShapeDtypeStruct((N,), jnp.float32),
        ),
        scratch_types=[
            two((_CHUNK,), jnp.int32),
            two((_CHUNK, 32), jnp.float32),
            two((_CHUNK,), jnp.float32),
            two((_CHUNK, 4), jnp.float32),
            two((_CHUNK, 32), jnp.float32),
            two((_CHUNK,), jnp.float32),
            [[pltpu.SemaphoreType.DMA] * _NS,
             [pltpu.SemaphoreType.DMA] * _NS],
            [pltpu.SemaphoreType.DMA, pltpu.SemaphoreType.DMA],
        ],
        compiler_params=pltpu.CompilerParams(use_tc_tiling_on_sc=False,
                                             needs_layout_passes=False),
    )
    def fk(ids_hbm, ph_hbm, tab_hbm, qb_out, qt_out, nrm_out,
           ids_v, tab_v, ph_v, qb_v, qt_v, nrm_v, gsem, osem):
        wid = lax.axis_index("s") * 2 + lax.axis_index("c")
        iota = lax.iota(jnp.int32, 16)
        w0 = wid * per_w

        def gather_copies(p):
            cps = []
            for jj in range(_NS):
                sl = pl.ds(jj * 128, 128)
                cps.append((tab_hbm.at[ids_v[p].at[sl]], tab_v[p].at[sl],
                            gsem[p][jj]))
            return cps

        def prep(t, p):
            g0 = w0 + t * _CHUNK
            pltpu.sync_copy(ids_hbm.at[pl.ds(g0, _CHUNK)], ids_v[p])
            pltpu.sync_copy(ph_hbm.at[pl.ds(g0, _CHUNK)], ph_v[p])
            for src, dst, sm in gather_copies(p):
                pltpu.async_copy(src, dst, sm)

        def wait_gathers(p):
            for src, dst, sm in gather_copies(p):
                pltpu.make_async_copy(src, dst, sm).wait()

        def out_copies(t, p):
            g0 = w0 + t * _CHUNK
            return [
                (qb_v[p], qb_out.at[pl.ds(g0, _CHUNK)], osem[p]),
                (qt_v[p], qt_out.at[pl.ds(g0, _CHUNK)], osem[p]),
                (nrm_v[p], nrm_out.at[pl.ds(g0, _CHUNK)], osem[p]),
            ]

        def wait_outs(t, p):
            for src, dst, sm in out_copies(t, p):
                pltpu.make_async_copy(src, dst, sm).wait()

        def compute(t, p):
            def group(k, c):
                rowg = k * 16 + iota
                b = [plsc.load_gather(tab_v[p],
                                      [rowg, jnp.full((16,), j, jnp.int32)])
                     for j in range(4)]
                nb2 = b[0] * b[0] + b[1] * b[1] + b[2] * b[2] + b[3] * b[3]
                invb = jnp.minimum(_rsqrt(nb2), 1.0 / EPS)
                q = [v * invb for v in b]
                x = ph_v[p][pl.ds(k * 16, 16)]
                tt = x * x
                s = _sin01(x, tt)
                cc = _cos01(tt)
                r4 = [cc * q[0] - s * q[1], s * q[0] + cc * q[1],
                      cc * q[2] - s * q[3], s * q[2] + cc * q[3]]
                diag = []
                dss = nb2 * 0.0
                for d in range(16):
                    col_a = (iota + d) & 15
                    col_b = col_a + 16
                    va = plsc.load_gather(tab_v[p], [rowg, col_a])
                    vb = plsc.load_gather(tab_v[p], [rowg, col_b])
                    diag.append((col_a, va))
                    diag.append((col_b, vb))
                    dss = dss + va * va + vb * vb
                n2 = (dss - nb2) + nb2 * invb * invb
                rr = _rsqrt(n2)
                inv = jnp.minimum(rr, 1.0 / EPS)
                n = n2 * rr
                for colv, v in diag:
                    plsc.store_scatter(qt_v[p], [rowg, colv], v * inv,
                                       mask=colv >= 4)
                for j in range(4):
                    cj = jnp.full((16,), j, jnp.int32)
                    plsc.store_scatter(qt_v[p], [rowg, cj], r4[j] * inv)
                    plsc.store_scatter(qb_v[p], [rowg, cj], r4[j])
                nrm_v[p][pl.ds(k * 16, 16)] = n * inv
                return c

            lax.fori_loop(0, _GROUPS, group, 0)
            for src, dst, sm in out_copies(t, p):
                pltpu.async_copy(src, dst, sm)

        prep(0, 0)

        def body2(u, carry):
            t0 = u * 2
            wait_gathers(0)
            prep(t0 + 1, 1)

            @pl.when(u > 0)
            def _():
                wait_outs(t0 - 2, 0)

            compute(t0, 0)
            wait_gathers(1)

            @pl.when(u + 1 < chunks // 2)
            def _():
                prep(t0 + 2, 0)

            @pl.when(u > 0)
            def _():
                wait_outs(t0 - 1, 1)

            compute(t0 + 1, 1)
            return carry

        lax.fori_loop(0, chunks // 2, body2, 0)
        wait_outs(chunks - 2, 0)
        wait_outs(chunks - 1, 1)

    return fk


def kernel(concept_ids, phase, base_table, context_table):
    B, L = concept_ids.shape
    N = B * L
    ids = concept_ids.astype(jnp.int32).reshape(N)
    table = jnp.concatenate([base_table, context_table], axis=1)
    qb, qt, nrm = _make_fused(N)(ids, phase.reshape(N), table)
    return (qb.reshape(B, L, 4), qt.reshape(B, L, 32),
            jnp.ones((), dtype=bool), nrm.reshape(B, L))


# speedup vs baseline: 1.1180x; 1.1180x over previous
import functools

import jax
import jax.numpy as jnp
from jax import lax
from jax.experimental import pallas as pl
from jax.experimental.pallas import tpu as pltpu
from jax.experimental.pallas import tpu_sc as plsc

EPS = 1e-8
_NW = 32
_CHUNK = 512
_GROUPS = _CHUNK // 16
_NS = _CHUNK // 128


def _rsqrt(x):
    y = plsc.bitcast(jnp.int32(0x5F3759DF) - (plsc.bitcast(x, jnp.int32) >> 1),
                     jnp.float32)
    y = y * (1.5 - 0.5 * x * y * y)
    y = y * (1.5 - 0.5 * x * y * y)
    y = y * (1.5 - 0.5 * x * y * y)
    return y


def _sin01(x, t):
    s = 2.7557319e-6 * t - 1.9841270e-4
    s = s * t + 8.3333333e-3
    s = s * t - 1.6666667e-1
    s = s * t + 1.0
    return x * s


def _cos01(t):
    c = -2.7557319e-7 * t + 2.4801587e-5
    c = c * t - 1.3888889e-3
    c = c * t + 4.1666667e-2
    c = c * t - 5.0e-1
    return c * t + 1.0


def _make_fused(N):
    per_w = N // _NW
    chunks = per_w // _CHUNK
    mesh = plsc.VectorSubcoreMesh(core_axis_name="c", subcore_axis_name="s")
    two = lambda shape, dt: [pltpu.VMEM(shape, dt), pltpu.VMEM(shape, dt)]

    @functools.partial(
        pl.kernel,
        mesh=mesh,
        out_type=(
            jax.ShapeDtypeStruct((N * 4,), jnp.float32),
            jax.ShapeDtypeStruct((N * 32,), jnp.float32),
            jax.ShapeDtypeStruct((N,), jnp.float32),
        ),
        scratch_types=[
            two((_CHUNK,), jnp.int32),
            two((_CHUNK, 32), jnp.float32),
            two((_CHUNK,), jnp.float32),
            two((_CHUNK * 4,), jnp.float32),
            two((_CHUNK * 32,), jnp.float32),
            two((_CHUNK,), jnp.float32),
            [[pltpu.SemaphoreType.DMA] * _NS,
             [pltpu.SemaphoreType.DMA] * _NS],
            [pltpu.SemaphoreType.DMA, pltpu.SemaphoreType.DMA],
        ],
        compiler_params=pltpu.CompilerParams(use_tc_tiling_on_sc=False,
                                             needs_layout_passes=False),
    )
    def fk(ids_hbm, ph_hbm, tab_hbm, qb_out, qt_out, nrm_out,
           ids_v, tab_v, ph_v, qb_v, qt_v, nrm_v, gsem, osem):
        wid = lax.axis_index("s") * 2 + lax.axis_index("c")
        iota = lax.iota(jnp.int32, 16)
        w0 = wid * per_w

        def gather_copies(p):
            cps = []
            for jj in range(_NS):
                sl = pl.ds(jj * 128, 128)
                cps.append((tab_hbm.at[ids_v[p].at[sl]], tab_v[p].at[sl],
                            gsem[p][jj]))
            return cps

        def prep(t, p):
            g0 = w0 + t * _CHUNK
            pltpu.sync_copy(ids_hbm.at[pl.ds(g0, _CHUNK)], ids_v[p])
            pltpu.sync_copy(ph_hbm.at[pl.ds(g0, _CHUNK)], ph_v[p])
            for src, dst, sm in gather_copies(p):
                pltpu.async_copy(src, dst, sm)

        def wait_gathers(p):
            for src, dst, sm in gather_copies(p):
                pltpu.make_async_copy(src, dst, sm).wait()

        def out_copies(t, p):
            g0 = w0 + t * _CHUNK
            return [
                (qb_v[p], qb_out.at[pl.ds(g0 * 4, _CHUNK * 4)], osem[p]),
                (qt_v[p], qt_out.at[pl.ds(g0 * 32, _CHUNK * 32)], osem[p]),
                (nrm_v[p], nrm_out.at[pl.ds(g0, _CHUNK)], osem[p]),
            ]

        def wait_outs(t, p):
            for src, dst, sm in out_copies(t, p):
                pltpu.make_async_copy(src, dst, sm).wait()

        def compute(t, p):
            def group(k, c):
                rowg = k * 16 + iota
                b = [plsc.load_gather(tab_v[p],
                                      [rowg, jnp.full((16,), j, jnp.int32)])
                     for j in range(4)]
                nb2 = b[0] * b[0] + b[1] * b[1] + b[2] * b[2] + b[3] * b[3]
                invb = jnp.minimum(_rsqrt(nb2), 1.0 / EPS)
                q = [v * invb for v in b]
                x = ph_v[p][pl.ds(k * 16, 16)]
                tt = x * x
                s = _sin01(x, tt)
                cc = _cos01(tt)
                r4 = [cc * q[0] - s * q[1], s * q[0] + cc * q[1],
                      cc * q[2] - s * q[3], s * q[2] + cc * q[3]]
                cx = [plsc.load_gather(tab_v[p],
                                       [rowg, jnp.full((16,), 4 + j, jnp.int32)])
                      for j in range(28)]
                n2 = (r4[0] * r4[0] + r4[1] * r4[1] + r4[2] * r4[2]
                      + r4[3] * r4[3])
                for v in cx:
                    n2 = n2 + v * v
                rr = _rsqrt(n2)
                inv = jnp.minimum(rr, 1.0 / EPS)
                n = n2 * rr
                row4 = rowg * 4
                row32 = rowg * 32
                for j in range(4):
                    plsc.store_scatter(qb_v[p], [row4 + j], r4[j])
                    plsc.store_scatter(qt_v[p], [row32 + j], r4[j] * inv)
                for j in range(28):
                    plsc.store_scatter(qt_v[p], [row32 + 4 + j], cx[j] * inv)
                nrm_v[p][pl.ds(k * 16, 16)] = n * inv
                return c

            lax.fori_loop(0, _GROUPS, group, 0)
            for src, dst, sm in out_copies(t, p):
                pltpu.async_copy(src, dst, sm)

        prep(0, 0)

        def body2(u, carry):
            t0 = u * 2
            wait_gathers(0)
            prep(t0 + 1, 1)

            @pl.when(u > 0)
            def _():
                wait_outs(t0 - 2, 0)

            compute(t0, 0)
            wait_gathers(1)

            @pl.when(u + 1 < chunks // 2)
            def _():
                prep(t0 + 2, 0)

            @pl.when(u > 0)
            def _():
                wait_outs(t0 - 1, 1)

            compute(t0 + 1, 1)
            return carry

        lax.fori_loop(0, chunks // 2, body2, 0)
        wait_outs(chunks - 2, 0)
        wait_outs(chunks - 1, 1)

    return fk


def kernel(concept_ids, phase, base_table, context_table):
    B, L = concept_ids.shape
    N = B * L
    ids = concept_ids.astype(jnp.int32).reshape(N)
    table = jnp.concatenate([base_table, context_table], axis=1)
    qb, qt, nrm = _make_fused(N)(ids, phase.reshape(N), table)
    return (qb.reshape(B, L, 4), qt.reshape(B, L, 32),
            jnp.ones((), dtype=bool), nrm.reshape(B, L))
